# baseline (speedup 1.0000x reference)
import functools

import jax
import jax.numpy as jnp
from jax import lax
from jax.experimental import pallas as pl
from jax.experimental.pallas import tpu as pltpu
from jax.experimental.pallas import tpu_sc as plsc

_NC = 2
_NS = 16
_L = 16

_BUF_WORDS = 65536
_UNROLL = 8


@functools.lru_cache(maxsize=None)
def _sc_zero_fill(n_words):
    n_workers = _NC * _NS
    per_worker = n_words // n_workers
    assert per_worker * n_workers == n_words
    spm_words = _NS * _BUF_WORDS
    assert per_worker % spm_words == 0
    n_copies = per_worker // spm_words

    mesh = plsc.VectorSubcoreMesh(
        core_axis_name="c", subcore_axis_name="s",
        num_cores=_NC, num_subcores=_NS,
    )

    @functools.partial(
        pl.kernel,
        out_type=jax.ShapeDtypeStruct((n_words,), jnp.float32),
        mesh=mesh,
        scratch_types=[
            pltpu.VMEM((_BUF_WORDS,), jnp.float32),
            pltpu.VMEM_SHARED((spm_words,), jnp.float32),
            pltpu.SemaphoreType.DMA,
        ],
    )
    def sc_zero(out_hbm, zbuf, spm, sem):
        cid = lax.axis_index("c")
        sid = lax.axis_index("s")
        wid = sid * _NC + cid
        zeros = jnp.zeros((_L,), jnp.float32)

        def zero_body(i, carry):
            base = i * (_L * _UNROLL)
            for u in range(_UNROLL):
                zbuf[pl.ds(base + u * _L, _L)] = zeros
            return carry

        lax.fori_loop(0, _BUF_WORDS // (_L * _UNROLL), zero_body, 0)

        pltpu.sync_copy(zbuf, spm.at[pl.ds(sid * _BUF_WORDS, _BUF_WORDS)])
        plsc.subcore_barrier()

        base = wid * per_worker
        copies = [
            pltpu.async_copy(
                spm,
                out_hbm.at[pl.ds(base + k * spm_words, spm_words)],
                sem,
            )
            for k in range(n_copies)
        ]
        for c in copies:
            c.wait()

    return sc_zero


def kernel(x):
    B, C = x.shape
    out = _sc_zero_fill(B * C)()
    return out.reshape(B, C)


# speedup vs baseline: 3.6428x; 3.6428x over previous
import functools

import jax
import jax.numpy as jnp
from jax import lax
from jax.experimental import pallas as pl
from jax.experimental.pallas import tpu as pltpu
from jax.experimental.pallas import tpu_sc as plsc

_NC = 2
_NS = 16
_L = 16

_ROWS_BUF = 32
_UNROLL = 8


@functools.lru_cache(maxsize=None)
def _sc_zero_fill(B, C):
    n_workers = _NC * _NS
    rows_per = B // n_workers
    assert rows_per * n_workers == B
    n_copies = rows_per // _ROWS_BUF
    assert n_copies * _ROWS_BUF == rows_per
    assert C % (_L * _UNROLL) == 0

    mesh = plsc.VectorSubcoreMesh(
        core_axis_name="c", subcore_axis_name="s",
        num_cores=_NC, num_subcores=_NS,
    )

    @functools.partial(
        pl.kernel,
        out_type=jax.ShapeDtypeStruct((B, C), jnp.float32),
        mesh=mesh,
        scratch_types=[
            pltpu.VMEM((_ROWS_BUF, C), jnp.float32),
            pltpu.SemaphoreType.DMA,
        ],
    )
    def sc_zero(out_hbm, zbuf, sem):
        wid = lax.axis_index("s") * _NC + lax.axis_index("c")
        zeros = jnp.zeros((_L,), jnp.float32)
        per_row = C // (_L * _UNROLL)

        def zero_body(i, carry):
            r = i // per_row
            cb = (i % per_row) * (_L * _UNROLL)
            for u in range(_UNROLL):
                zbuf[r, pl.ds(cb + u * _L, _L)] = zeros
            return carry

        lax.fori_loop(0, _ROWS_BUF * per_row, zero_body, 0)

        base = wid * rows_per
        copies = [
            pltpu.async_copy(
                zbuf,
                out_hbm.at[pl.ds(base + k * _ROWS_BUF, _ROWS_BUF)],
                sem,
            )
            for k in range(n_copies)
        ]
        for c in copies:
            c.wait()

    return sc_zero


def kernel(x):
    B, C = x.shape
    return _sc_zero_fill(B, C)()


# speedup vs baseline: 3.7431x; 1.0275x over previous
import functools

import jax
import jax.numpy as jnp
from jax import lax
from jax.experimental import pallas as pl
from jax.experimental.pallas import tpu as pltpu
from jax.experimental.pallas import tpu_sc as plsc

_NC = 2
_NS = 16
_L = 16

_ROWS_BUF = 32
_UNROLL = 8


@functools.lru_cache(maxsize=None)
def _sc_zero_fill(B, C):
    n_workers = _NC * _NS
    rows_per = B // n_workers
    assert rows_per * n_workers == B
    n_copies = rows_per // _ROWS_BUF
    assert n_copies * _ROWS_BUF == rows_per
    assert C % (_L * _UNROLL) == 0

    mesh = plsc.VectorSubcoreMesh(
        core_axis_name="c", subcore_axis_name="s",
        num_cores=_NC, num_subcores=_NS,
    )

    @functools.partial(
        pl.kernel,
        out_type=jax.ShapeDtypeStruct((B, C), jnp.float32),
        mesh=mesh,
        scratch_types=[
            pltpu.VMEM((_ROWS_BUF, C), jnp.float32),
            pltpu.SemaphoreType.DMA,
        ],
    )
    def sc_zero(out_hbm, zbuf, sem):
        wid = lax.axis_index("s") * _NC + lax.axis_index("c")
        zeros = jnp.zeros((_L,), jnp.float32)
        per_row = C // (_L * _UNROLL)
        base = wid * rows_per

        def fill_rows(r0, r1):
            def zero_body(i, carry):
                r = r0 + i // per_row
                cb = (i % per_row) * (_L * _UNROLL)
                for u in range(_UNROLL):
                    zbuf[r, pl.ds(cb + u * _L, _L)] = zeros
                return carry

            lax.fori_loop(0, (r1 - r0) * per_row, zero_body, 0)

        q = _ROWS_BUF // 4
        fill_rows(0, q)
        first = [
            pltpu.async_copy(
                zbuf.at[pl.ds(0, q)],
                out_hbm.at[pl.ds(base + j * q, q)],
                sem,
            )
            for j in range(4)
        ]
        fill_rows(q, _ROWS_BUF)
        rest = [
            pltpu.async_copy(
                zbuf,
                out_hbm.at[pl.ds(base + (k + 1) * _ROWS_BUF, _ROWS_BUF)],
                sem,
            )
            for k in range(n_copies - 1)
        ]
        for c in first + rest:
            c.wait()

    return sc_zero


def kernel(x):
    B, C = x.shape
    return _sc_zero_fill(B, C)()
